# Initial kernel scaffold; baseline (speedup 1.0000x reference)
import jax
import jax.numpy as jnp
from jax.experimental import pallas as pl


def kernel(x, edge_index, edge_attr, batch, params):
    raise NotImplementedError("write your pallas kernel here")


# speedup vs baseline: 15.2905x; 15.2905x over previous
import functools

import jax
import jax.numpy as jnp
from jax import lax
from jax.experimental import pallas as pl
from jax.experimental.pallas import tpu as pltpu
from jax.experimental.pallas import tpu_sc as plsc

N = 10000
E = 320000
D = 128
H = 128
G = 64
L_MOL = 2

NC, NS, LN = 2, 16, 16
NW = NC * NS
EW = E // NW
CB = 80
NB = EW // CB
BE = 3200
NBE = E // BE

F32 = jnp.float32


def _leaky(x, s):
    return jnp.where(x >= 0, x, s * x)


def _sigmoid(x):
    return 1.0 / (1.0 + jnp.exp(-x))


def _elu(x):
    return jnp.where(x > 0, x, jnp.exp(jnp.minimum(x, 0.0)) - 1.0)


def _gru(inp, h, Wih, Whh, bih, bhh):
    gi = jnp.dot(inp, Wih.T, preferred_element_type=F32) + bih
    gh = jnp.dot(h, Whh.T, preferred_element_type=F32) + bhh
    r = _sigmoid(gi[:, :H] + gh[:, :H])
    z = _sigmoid(gi[:, H:2 * H] + gh[:, H:2 * H])
    n = jnp.tanh(gi[:, 2 * H:] + r * gh[:, 2 * H:])
    return (1.0 - z) * n + z * h


def _t1_body(x_ref, w_ref, b_ref, w1_ref, attr_ref,
             x1_ref, xproj_ref, ar_ref, armax_ref):
    x1 = _leaky(jnp.dot(x_ref[...], w_ref[...].T, preferred_element_type=F32)
                + b_ref[...], 0.01)
    x1_ref[...] = x1
    xproj_ref[...] = jnp.dot(x1, w1_ref[...].T, preferred_element_type=F32)
    ar = jnp.dot(x1, attr_ref[...], preferred_element_type=F32)
    ar_ref[...] = ar
    armax_ref[...] = jnp.max(ar).reshape(1, 1)


def _t1(x, lin1_W, lin1_b, W1, att_r):
    return pl.pallas_call(
        _t1_body,
        out_shape=[jax.ShapeDtypeStruct((N, D), F32),
                   jax.ShapeDtypeStruct((N, D), F32),
                   jax.ShapeDtypeStruct((N, 1), F32),
                   jax.ShapeDtypeStruct((1, 1), F32)],
    )(x, lin1_W, lin1_b, W1, att_r.reshape(D, 1))


def _t2_body(ea_ref, gsrc_ref, w2_ref, bg_ref, wl2_ref, attl_ref,
             mmsg_ref, xjatt_ref, xjattmax_ref):
    i = pl.program_id(0)
    xj = _leaky(gsrc_ref[...]
                + jnp.dot(ea_ref[...], w2_ref[...].T, preferred_element_type=F32)
                + bg_ref[...], 0.01)
    mmsg_ref[...] = jnp.dot(xj, wl2_ref[...].T, preferred_element_type=F32)
    xa = jnp.dot(xj, attl_ref[...], preferred_element_type=F32)
    xjatt_ref[...] = xa

    @pl.when(i == 0)
    def _():
        xjattmax_ref[...] = jnp.full((1, 1), -jnp.inf, F32)
    xjattmax_ref[...] = jnp.maximum(xjattmax_ref[...], jnp.max(xa).reshape(1, 1))


def _t2(edge_attr, gsrc, W2, bg, Wl2, att_l):
    return pl.pallas_call(
        _t2_body,
        grid=(NBE,),
        in_specs=[pl.BlockSpec((BE, D), lambda i: (i, 0)),
                  pl.BlockSpec((BE, D), lambda i: (i, 0)),
                  pl.BlockSpec((D, D), lambda i: (0, 0)),
                  pl.BlockSpec((1, D), lambda i: (0, 0)),
                  pl.BlockSpec((D, D), lambda i: (0, 0)),
                  pl.BlockSpec((D, 1), lambda i: (0, 0))],
        out_specs=[pl.BlockSpec((BE, D), lambda i: (i, 0)),
                   pl.BlockSpec((BE, 1), lambda i: (i, 0)),
                   pl.BlockSpec((1, 1), lambda i: (0, 0))],
        out_shape=[jax.ShapeDtypeStruct((E, D), F32),
                   jax.ShapeDtypeStruct((E, 1), F32),
                   jax.ShapeDtypeStruct((1, 1), F32)],
    )(edge_attr, gsrc, W2, bg.reshape(1, D), Wl2, att_l.reshape(D, 1))


CBG = 400
NBG = EW // CBG


def _scg_body(tab_hbm, idx_hbm, out_hbm, idx_v, rows_v, sem):
    wid = lax.axis_index("s") * NC + lax.axis_index("c")
    base0 = wid * EW

    def blk(bi, _):
        base = base0 + bi * CBG
        pltpu.sync_copy(idx_hbm.at[pl.ds(base, CBG)], idx_v)
        pltpu.async_copy(tab_hbm.at[idx_v], rows_v, sem).wait()
        pltpu.sync_copy(rows_v, out_hbm.at[pl.ds(base, CBG)])
        return 0

    lax.fori_loop(0, NBG, blk, 0, unroll=False)


def _sc_gather(table, idx):
    mesh = plsc.VectorSubcoreMesh(core_axis_name="c", subcore_axis_name="s",
                                  num_cores=NC, num_subcores=NS)
    return pl.kernel(
        _scg_body,
        out_type=jax.ShapeDtypeStruct((E, D), F32),
        mesh=mesh,
        scratch_types=[pltpu.VMEM((CBG,), jnp.int32),
                       pltpu.VMEM((CBG, D), F32),
                       pltpu.SemaphoreType.DMA],
    )(table, idx)


def _sc_agg_body(slope, gather_rows,
                 rowsrc_hbm, xa_hbm, src_hbm, dst_hbm, ars_hbm, ard_hbm,
                 b_hbm, zeros_hbm,
                 sacc_hbm, hacc_hbm,
                 ars_v, ard_v, spart, src_v, dst_v, xa_v, rows_v, b_v,
                 shared, sem):
    c = lax.axis_index("c")
    s = lax.axis_index("s")
    wid = s * NC + c
    base0 = wid * EW

    pltpu.sync_copy(ard_hbm, ard_v)
    if gather_rows:
        pltpu.sync_copy(ars_hbm, ars_v)
    pltpu.sync_copy(b_hbm, b_v)

    def zf(i, _):
        spart[pl.ds(i * LN, LN)] = jnp.zeros((LN,), F32)
        return 0
    lax.fori_loop(0, N // LN, zf, 0, unroll=False)

    @pl.when(s == 0)
    def _():
        pltpu.sync_copy(zeros_hbm, shared)
    plsc.subcore_barrier()

    bv = b_v[...]

    def blk(bi, _):
        base = base0 + bi * CB
        pltpu.sync_copy(dst_hbm.at[pl.ds(base, CB)], dst_v)
        if gather_rows:
            pltpu.sync_copy(src_hbm.at[pl.ds(base, CB)], src_v)
            pltpu.async_copy(rowsrc_hbm.at[src_v], rows_v, sem).wait()
        else:
            pltpu.sync_copy(xa_hbm.at[pl.ds(base, CB)], xa_v)
            pltpu.sync_copy(rowsrc_hbm.at[pl.ds(base, CB)], rows_v)

        def inner(j, _):
            dvec = dst_v[pl.ds(j * LN, LN)]
            adv = plsc.load_gather(ard_v, [dvec])
            if gather_rows:
                svec = src_v[pl.ds(j * LN, LN)]
                asv = plsc.load_gather(ars_v, [svec])
                t = asv + adv
            else:
                t = xa_v[pl.ds(j * LN, LN)] + adv
            logit = jnp.where(t >= 0, t, slope * t)
            e = jnp.exp(logit - bv)
            plsc.addupdate_scatter(spart, [dvec], e)
            for k in range(LN):
                wv = jnp.zeros((LN,), F32) + e[k]
                r = j * LN + k
                for q in range(D // LN):
                    rows_v[r, pl.ds(q * LN, LN)] = (
                        rows_v[r, pl.ds(q * LN, LN)] * wv)
            return 0
        lax.fori_loop(0, CB // LN, inner, 0, unroll=False)

        pltpu.sync_copy(rows_v, shared.at[dst_v], add=True)
        return 0

    lax.fori_loop(0, NB, blk, 0, unroll=False)

    pltpu.sync_copy(spart, sacc_hbm.at[wid])
    plsc.subcore_barrier()

    @pl.when(s == 0)
    def _():
        pltpu.sync_copy(shared, hacc_hbm.at[c])


def _sc_aggregate(slope, gather_rows, rowsrc, xa, src, dst, ars, ard, bound):
    mesh = plsc.VectorSubcoreMesh(core_axis_name="c", subcore_axis_name="s",
                                  num_cores=NC, num_subcores=NS)
    body = functools.partial(_sc_agg_body, slope, gather_rows)
    return pl.kernel(
        body,
        out_type=[jax.ShapeDtypeStruct((NW, N), F32),
                  jax.ShapeDtypeStruct((NC, N, D), F32)],
        mesh=mesh,
        compiler_params=pltpu.CompilerParams(use_tc_tiling_on_sc=False,
                                             needs_layout_passes=False),
        scratch_types=[pltpu.VMEM((N if gather_rows else LN,), F32),
                       pltpu.VMEM((N,), F32),
                       pltpu.VMEM((N,), F32),
                       pltpu.VMEM((CB if gather_rows else LN,), jnp.int32),
                       pltpu.VMEM((CB,), jnp.int32),
                       pltpu.VMEM((CB if not gather_rows else LN,), F32),
                       pltpu.VMEM((CB, D), F32),
                       pltpu.VMEM((LN,), F32),
                       pltpu.VMEM_SHARED((N, D), F32),
                       pltpu.SemaphoreType.DMA],
    )(rowsrc, xa, src, dst, ars, ard, bound,
      jnp.zeros((N, D), F32))


def _t3_body(hacc_ref, sacc_ref, gbias_ref, x1_ref,
             wih_ref, whh_ref, bih_ref, bhh_ref,
             gat1w_ref, attsrc_ref, attdst_ref,
             x2_ref, hsrc_ref, as_ref, ad_ref, eself_ref, b2_ref):
    ssum = jnp.sum(sacc_ref[...], axis=0)[:, None]
    hsum = (hacc_ref[0] + hacc_ref[1]) / (ssum + 1e-16)
    h = _elu(hsum + gbias_ref[...])
    x2 = jnp.maximum(_gru(h, x1_ref[...], wih_ref[...], whh_ref[...],
                          bih_ref[...], bhh_ref[...]), 0.0)
    x2_ref[...] = x2
    hsrc = jnp.dot(x2, gat1w_ref[...].T, preferred_element_type=F32)
    hsrc_ref[...] = hsrc
    as_ = jnp.dot(hsrc, attsrc_ref[...], preferred_element_type=F32)
    ad = jnp.dot(hsrc, attdst_ref[...], preferred_element_type=F32)
    as_ref[...] = as_
    ad_ref[...] = ad
    b2 = _leaky(jnp.max(as_) + jnp.max(ad), 0.2)
    b2_ref[...] = b2.reshape(1, 1)
    eself_ref[...] = jnp.exp(_leaky(as_ + ad, 0.2) - b2)


def _t3(hacc, sacc, gate_bias, x1, gru0, gat1_W, att_src, att_dst):
    return pl.pallas_call(
        _t3_body,
        out_shape=[jax.ShapeDtypeStruct((N, D), F32),
                   jax.ShapeDtypeStruct((N, D), F32),
                   jax.ShapeDtypeStruct((N, 1), F32),
                   jax.ShapeDtypeStruct((N, 1), F32),
                   jax.ShapeDtypeStruct((N, 1), F32),
                   jax.ShapeDtypeStruct((1, 1), F32)],
    )(hacc, sacc, gate_bias.reshape(1, D), x1, *gru0, gat1_W,
      att_src.reshape(D, 1), att_dst.reshape(D, 1))


def _t4a_body(hacc_ref, sacc_ref, eself_ref, hsrc_ref, gbias_ref, x2_ref,
              wih_ref, whh_ref, bih_ref, bhh_ref, x3_ref):
    s2 = jnp.sum(sacc_ref[...], axis=0)[:, None] + eself_ref[...]
    hun = hacc_ref[0] + hacc_ref[1] + eself_ref[...] * hsrc_ref[...]
    h = _elu(hun / (s2 + 1e-16) + gbias_ref[...])
    x3_ref[...] = jnp.maximum(_gru(h, x2_ref[...], wih_ref[...], whh_ref[...],
                                   bih_ref[...], bhh_ref[...]), 0.0)


def _t4a(hacc2, sacc2, eself, hsrc, gat1_bias, x2, gru1):
    return pl.pallas_call(
        _t4a_body,
        out_shape=jax.ShapeDtypeStruct((N, D), F32),
    )(hacc2, sacc2, eself, hsrc, gat1_bias.reshape(1, D), x2, *gru1)


def _t4b_body(x3_ref, batch_ref, wsrc_ref, wdst_ref, attsrc_ref, attdst_ref,
              mbias_ref, wih_ref, whh_ref, bih_ref, bhh_ref, out_ref):
    x3 = x3_ref[...]
    seg = lax.broadcasted_iota(jnp.int32, (G, N), 0)
    mask = (batch_ref[...] == seg).astype(F32)
    out = jnp.maximum(jnp.dot(mask, x3, preferred_element_type=F32), 0.0)
    for _ in range(L_MOL):
        hs = jnp.dot(x3, wsrc_ref[...].T, preferred_element_type=F32)
        hd = jnp.dot(out, wdst_ref[...].T, preferred_element_type=F32)
        asrc = jnp.dot(hs, attsrc_ref[...], preferred_element_type=F32)
        adst = jnp.dot(hd, attdst_ref[...], preferred_element_type=F32)
        adst_n = jnp.dot(adst.T, mask, preferred_element_type=F32).T
        logits = _leaky(asrc + adst_n, 0.2)
        lg = jnp.where(mask > 0, logits.T, -jnp.inf)
        m = jnp.max(lg, axis=1, keepdims=True)
        m = jnp.where(jnp.isfinite(m), m, 0.0)
        m_n = jnp.dot(m.T, mask, preferred_element_type=F32).T
        ee = jnp.exp(logits - m_n)
        sseg = jnp.dot(mask, ee, preferred_element_type=F32)
        s_n = jnp.dot(sseg.T, mask, preferred_element_type=F32).T
        alpha = ee / (s_n + 1e-16)
        hm = _elu(jnp.dot(mask, hs * alpha, preferred_element_type=F32)
                  + mbias_ref[...])
        out = jnp.maximum(_gru(hm, out, wih_ref[...], whh_ref[...],
                               bih_ref[...], bhh_ref[...]), 0.0)
    out_ref[...] = out


def _t4b(x3, batch, p):
    return pl.pallas_call(
        _t4b_body,
        out_shape=jax.ShapeDtypeStruct((G, D), F32),
    )(x3, batch.reshape(1, N), p['mol_W_src'], p['mol_W_dst'],
      p['mol_att_src'].reshape(D, 1), p['mol_att_dst'].reshape(D, 1),
      p['mol_bias'].reshape(1, D),
      p['mol_gru_Wih'], p['mol_gru_Whh'],
      p['mol_gru_bih'].reshape(1, 3 * H), p['mol_gru_bhh'].reshape(1, 3 * H))


def kernel(x, edge_index, edge_attr, batch, params):
    p = params
    src, dst = edge_index[0], edge_index[1]
    W1 = p['gate_lin1_W'][:, :H]
    W2 = p['gate_lin1_W'][:, H:]

    x1, xproj, ar, armax = _t1(x, p['lin1_W'], p['lin1_b'].reshape(1, D),
                               W1, p['gate_att_r'])

    gsrc = _sc_gather(xproj, src)

    mmsg, xjatt, xjattmax = _t2(edge_attr, gsrc, W2, p['gate_lin1_b'],
                                p['gate_lin2_W'], p['gate_att_l'])

    b1 = _leaky(xjattmax[0, 0] + armax[0, 0], 0.01)
    sacc, hacc = _sc_aggregate(
        0.01, False, mmsg, xjatt.reshape(E), src, dst,
        ar.reshape(N), ar.reshape(N), jnp.full((LN,), b1, F32))

    gru0 = (p['gru0_Wih'], p['gru0_Whh'],
            p['gru0_bih'].reshape(1, 3 * H), p['gru0_bhh'].reshape(1, 3 * H))
    x2, hsrc, as_, ad, eself, b2 = _t3(hacc, sacc, p['gate_bias'], x1, gru0,
                                       p['gat1_W'], p['gat1_att_src'],
                                       p['gat1_att_dst'])

    sacc2, hacc2 = _sc_aggregate(
        0.2, True, hsrc, xjatt.reshape(E), src, dst,
        as_.reshape(N), ad.reshape(N),
        jnp.full((LN,), b2[0, 0], F32))

    gru1 = (p['gru1_Wih'], p['gru1_Whh'],
            p['gru1_bih'].reshape(1, 3 * H), p['gru1_bhh'].reshape(1, 3 * H))
    x3 = _t4a(hacc2, sacc2, eself, hsrc, p['gat1_bias'], x2, gru1)

    return _t4b(x3, batch, p)


# speedup vs baseline: 19.5664x; 1.2796x over previous
import functools

import jax
import jax.numpy as jnp
from jax import lax
from jax.experimental import pallas as pl
from jax.experimental.pallas import tpu as pltpu
from jax.experimental.pallas import tpu_sc as plsc

N = 10000
E = 320000
D = 128
H = 128
G = 64
L_MOL = 2

NC, NS, LN = 2, 16, 16
NW = NC * NS
EW = E // NW
CB = 80
NB = EW // CB
BE = 3200
NBE = E // BE

F32 = jnp.float32


def _leaky(x, s):
    return jnp.where(x >= 0, x, s * x)


def _sigmoid(x):
    return 1.0 / (1.0 + jnp.exp(-x))


def _elu(x):
    return jnp.where(x > 0, x, jnp.exp(jnp.minimum(x, 0.0)) - 1.0)


def _gru(inp, h, Wih, Whh, bih, bhh):
    gi = jnp.dot(inp, Wih.T, preferred_element_type=F32) + bih
    gh = jnp.dot(h, Whh.T, preferred_element_type=F32) + bhh
    r = _sigmoid(gi[:, :H] + gh[:, :H])
    z = _sigmoid(gi[:, H:2 * H] + gh[:, H:2 * H])
    n = jnp.tanh(gi[:, 2 * H:] + r * gh[:, 2 * H:])
    return (1.0 - z) * n + z * h


def _t1_body(x_ref, w_ref, b_ref, w1_ref, attr_ref,
             x1_ref, xproj_ref, ar_ref, armax_ref):
    x1 = _leaky(jnp.dot(x_ref[...], w_ref[...].T, preferred_element_type=F32)
                + b_ref[...], 0.01)
    x1_ref[...] = x1
    xproj_ref[...] = jnp.dot(x1, w1_ref[...].T, preferred_element_type=F32)
    ar = jnp.dot(x1, attr_ref[...], preferred_element_type=F32)
    ar_ref[...] = ar
    armax_ref[...] = jnp.max(ar).reshape(1, 1)


def _t1(x, lin1_W, lin1_b, W1, att_r):
    return pl.pallas_call(
        _t1_body,
        out_shape=[jax.ShapeDtypeStruct((N, D), F32),
                   jax.ShapeDtypeStruct((N, D), F32),
                   jax.ShapeDtypeStruct((N, 1), F32),
                   jax.ShapeDtypeStruct((1, 1), F32)],
    )(x, lin1_W, lin1_b, W1, att_r.reshape(D, 1))


def _t2_body(ea_ref, gsrc_ref, w2_ref, bg_ref, wl2_ref, attl_ref,
             mmsg_ref, xjatt_ref, xjattmax_ref):
    i = pl.program_id(0)
    xj = _leaky(gsrc_ref[...]
                + jnp.dot(ea_ref[...], w2_ref[...].T, preferred_element_type=F32)
                + bg_ref[...], 0.01)
    mmsg_ref[...] = jnp.dot(xj, wl2_ref[...].T, preferred_element_type=F32)
    xa = jnp.dot(xj, attl_ref[...], preferred_element_type=F32)
    xjatt_ref[...] = xa

    @pl.when(i == 0)
    def _():
        xjattmax_ref[...] = jnp.full((1, 1), -jnp.inf, F32)
    xjattmax_ref[...] = jnp.maximum(xjattmax_ref[...], jnp.max(xa).reshape(1, 1))


def _t2(edge_attr, gsrc, W2, bg, Wl2, att_l):
    return pl.pallas_call(
        _t2_body,
        grid=(NBE,),
        in_specs=[pl.BlockSpec((BE, D), lambda i: (i, 0)),
                  pl.BlockSpec((BE, D), lambda i: (i, 0)),
                  pl.BlockSpec((D, D), lambda i: (0, 0)),
                  pl.BlockSpec((1, D), lambda i: (0, 0)),
                  pl.BlockSpec((D, D), lambda i: (0, 0)),
                  pl.BlockSpec((D, 1), lambda i: (0, 0))],
        out_specs=[pl.BlockSpec((BE, D), lambda i: (i, 0)),
                   pl.BlockSpec((BE, 1), lambda i: (i, 0)),
                   pl.BlockSpec((1, 1), lambda i: (0, 0))],
        out_shape=[jax.ShapeDtypeStruct((E, D), F32),
                   jax.ShapeDtypeStruct((E, 1), F32),
                   jax.ShapeDtypeStruct((1, 1), F32)],
    )(edge_attr, gsrc, W2, bg.reshape(1, D), Wl2, att_l.reshape(D, 1))


CBG = 400
NBG = EW // CBG


def _scg_body(tab_hbm, idx_hbm, out_hbm,
              idx_v0, idx_v1, rows_v0, rows_v1,
              semi0, semi1, semg, semo0, semo1):
    wid = lax.axis_index("s") * NC + lax.axis_index("c")
    base0 = wid * EW
    idx_b = (idx_v0, idx_v1)
    rows_b = (rows_v0, rows_v1)
    semi = (semi0, semi1)
    semo = (semo0, semo1)

    for b in range(2):
        pltpu.async_copy(idx_hbm.at[pl.ds(base0 + b * CBG, CBG)],
                         idx_b[b], semi[b])

    def block(g, b, first, prefetch):
        if not first:
            pltpu.make_async_copy(
                rows_b[b], out_hbm.at[pl.ds(base0 + (g - 2) * CBG, CBG)],
                semo[b]).wait()
        pltpu.make_async_copy(idx_hbm.at[pl.ds(base0 + g * CBG, CBG)],
                              idx_b[b], semi[b]).wait()
        pltpu.async_copy(tab_hbm.at[idx_b[b]], rows_b[b], semg).wait()

        if prefetch == "dynamic":
            @pl.when(g + 2 <= NBG - 1)
            def _():
                pltpu.async_copy(idx_hbm.at[pl.ds(base0 + (g + 2) * CBG, CBG)],
                                 idx_b[b], semi[b])
        elif prefetch:
            pltpu.async_copy(idx_hbm.at[pl.ds(base0 + (g + 2) * CBG, CBG)],
                             idx_b[b], semi[b])
        pltpu.async_copy(rows_b[b], out_hbm.at[pl.ds(base0 + g * CBG, CBG)],
                         semo[b])

    block(0, 0, True, True)
    block(1, 1, True, True)

    def pair(i, _):
        block(2 * i, 0, False, "dynamic")
        block(2 * i + 1, 1, False, "dynamic")
        return 0
    lax.fori_loop(1, NBG // 2, pair, 0, unroll=False)
    block(NBG - 1, 0, False, False)

    pltpu.make_async_copy(
        rows_b[0], out_hbm.at[pl.ds(base0 + (NBG - 1) * CBG, CBG)],
        semo[0]).wait()
    pltpu.make_async_copy(
        rows_b[1], out_hbm.at[pl.ds(base0 + (NBG - 2) * CBG, CBG)],
        semo[1]).wait()


def _sc_gather(table, idx):
    mesh = plsc.VectorSubcoreMesh(core_axis_name="c", subcore_axis_name="s",
                                  num_cores=NC, num_subcores=NS)
    return pl.kernel(
        _scg_body,
        out_type=jax.ShapeDtypeStruct((E, D), F32),
        mesh=mesh,
        scratch_types=[pltpu.VMEM((CBG,), jnp.int32),
                       pltpu.VMEM((CBG,), jnp.int32),
                       pltpu.VMEM((CBG, D), F32),
                       pltpu.VMEM((CBG, D), F32),
                       pltpu.SemaphoreType.DMA,
                       pltpu.SemaphoreType.DMA,
                       pltpu.SemaphoreType.DMA,
                       pltpu.SemaphoreType.DMA,
                       pltpu.SemaphoreType.DMA],
    )(table, idx)


def _sc_agg_body(slope, gather_rows,
                 rowsrc_hbm, aux_hbm, dst_hbm, ars_hbm, ard_hbm,
                 b_hbm, zeros_hbm, zeros1_hbm,
                 sacc_hbm, hacc_hbm,
                 ars_v, ard_v, b_v,
                 dst_v0, dst_v1, aux_v0, aux_v1, e_v0, e_v1,
                 rows_v0, rows_v1,
                 shared, shared_s,
                 semi0, semi1, semg, sems0, sems1):
    c = lax.axis_index("c")
    s = lax.axis_index("s")
    wid = s * NC + c
    base0 = wid * EW
    dst_b = (dst_v0, dst_v1)
    aux_b = (aux_v0, aux_v1)
    e_b = (e_v0, e_v1)
    rows_b = (rows_v0, rows_v1)
    semi = (semi0, semi1)
    sems = (sems0, sems1)

    pltpu.sync_copy(ard_hbm, ard_v)
    if gather_rows:
        pltpu.sync_copy(ars_hbm, ars_v)
    pltpu.sync_copy(b_hbm, b_v)

    @pl.when(s == 0)
    def _():
        pltpu.sync_copy(zeros_hbm, shared)
        pltpu.sync_copy(zeros1_hbm, shared_s)
    plsc.subcore_barrier()

    bv = b_v[...]

    def block(g, b, first):
        if not first:
            pltpu.make_async_copy(rows_b[b], shared.at[dst_b[b]],
                                  sems[b]).wait()
            pltpu.make_async_copy(e_b[b], shared_s.at[dst_b[b]],
                                  sems[b]).wait()
        base = base0 + g * CB
        cps = [pltpu.async_copy(dst_hbm.at[pl.ds(base, CB)], dst_b[b],
                                semi[b]),
               pltpu.async_copy(aux_hbm.at[pl.ds(base, CB)], aux_b[b],
                                semi[b])]
        if not gather_rows:
            cps.append(pltpu.async_copy(rowsrc_hbm.at[pl.ds(base, CB)],
                                        rows_b[b], semi[b]))
        for cp in cps:
            cp.wait()
        if gather_rows:
            pltpu.async_copy(rowsrc_hbm.at[aux_b[b]], rows_b[b], semg).wait()

        def inner(j, _):
            dvec = dst_b[b][pl.ds(j * LN, LN)]
            adv = plsc.load_gather(ard_v, [dvec])
            av = aux_b[b][pl.ds(j * LN, LN)]
            if gather_rows:
                t = plsc.load_gather(ars_v, [av]) + adv
            else:
                t = av + adv
            logit = jnp.where(t >= 0, t, slope * t)
            e = jnp.exp(logit - bv)
            e_b[b][pl.ds(j * LN, LN)] = e
            for k in range(LN):
                wv = jnp.zeros((LN,), F32) + e[k]
                r = j * LN + k
                for q in range(D // LN):
                    rows_b[b][r, pl.ds(q * LN, LN)] = (
                        rows_b[b][r, pl.ds(q * LN, LN)] * wv)
            return 0
        lax.fori_loop(0, CB // LN, inner, 0, unroll=False)

        pltpu.async_copy(rows_b[b], shared.at[dst_b[b]], sems[b], add=True)
        pltpu.async_copy(e_b[b], shared_s.at[dst_b[b]], sems[b], add=True)

    block(0, 0, True)
    block(1, 1, True)

    def pairs(i, _):
        block(2 * i, 0, False)
        block(2 * i + 1, 1, False)
        return 0
    lax.fori_loop(1, NB // 2, pairs, 0, unroll=False)
    block(NB - 1, 0, False)

    for b in range(2):
        pltpu.make_async_copy(rows_b[b], shared.at[dst_b[b]],
                              sems[b]).wait()
        pltpu.make_async_copy(e_b[b], shared_s.at[dst_b[b]],
                              sems[b]).wait()

    plsc.subcore_barrier()

    @pl.when(s == 0)
    def _():
        pltpu.sync_copy(shared, hacc_hbm.at[c])
        pltpu.sync_copy(shared_s, sacc_hbm.at[c])


def _sc_aggregate(slope, gather_rows, rowsrc, aux, dst, ars, ard, bound):
    mesh = plsc.VectorSubcoreMesh(core_axis_name="c", subcore_axis_name="s",
                                  num_cores=NC, num_subcores=NS)
    body = functools.partial(_sc_agg_body, slope, gather_rows)
    aux_dt = jnp.int32 if gather_rows else F32
    return pl.kernel(
        body,
        out_type=[jax.ShapeDtypeStruct((NC, N), F32),
                  jax.ShapeDtypeStruct((NC, N, D), F32)],
        mesh=mesh,
        compiler_params=pltpu.CompilerParams(use_tc_tiling_on_sc=False,
                                             needs_layout_passes=False),
        scratch_types=[pltpu.VMEM((N if gather_rows else LN,), F32),
                       pltpu.VMEM((N,), F32),
                       pltpu.VMEM((LN,), F32),
                       pltpu.VMEM((CB,), jnp.int32),
                       pltpu.VMEM((CB,), jnp.int32),
                       pltpu.VMEM((CB,), aux_dt),
                       pltpu.VMEM((CB,), aux_dt),
                       pltpu.VMEM((CB,), F32),
                       pltpu.VMEM((CB,), F32),
                       pltpu.VMEM((CB, D), F32),
                       pltpu.VMEM((CB, D), F32),
                       pltpu.VMEM_SHARED((N, D), F32),
                       pltpu.VMEM_SHARED((N,), F32),
                       pltpu.SemaphoreType.DMA,
                       pltpu.SemaphoreType.DMA,
                       pltpu.SemaphoreType.DMA,
                       pltpu.SemaphoreType.DMA,
                       pltpu.SemaphoreType.DMA],
    )(rowsrc, aux, dst, ars, ard, bound,
      jnp.zeros((N, D), F32), jnp.zeros((N,), F32))


def _t3_body(hacc_ref, sacc_ref, gbias_ref, x1_ref,
             wih_ref, whh_ref, bih_ref, bhh_ref,
             gat1w_ref, attsrc_ref, attdst_ref,
             x2_ref, hsrc_ref, as_ref, ad_ref, eself_ref, b2_ref):
    ssum = jnp.sum(sacc_ref[...], axis=0)[:, None]
    hsum = (hacc_ref[0] + hacc_ref[1]) / (ssum + 1e-16)
    h = _elu(hsum + gbias_ref[...])
    x2 = jnp.maximum(_gru(h, x1_ref[...], wih_ref[...], whh_ref[...],
                          bih_ref[...], bhh_ref[...]), 0.0)
    x2_ref[...] = x2
    hsrc = jnp.dot(x2, gat1w_ref[...].T, preferred_element_type=F32)
    hsrc_ref[...] = hsrc
    as_ = jnp.dot(hsrc, attsrc_ref[...], preferred_element_type=F32)
    ad = jnp.dot(hsrc, attdst_ref[...], preferred_element_type=F32)
    as_ref[...] = as_
    ad_ref[...] = ad
    b2 = _leaky(jnp.max(as_) + jnp.max(ad), 0.2)
    b2_ref[...] = b2.reshape(1, 1)
    eself_ref[...] = jnp.exp(_leaky(as_ + ad, 0.2) - b2)


def _t3(hacc, sacc, gate_bias, x1, gru0, gat1_W, att_src, att_dst):
    return pl.pallas_call(
        _t3_body,
        out_shape=[jax.ShapeDtypeStruct((N, D), F32),
                   jax.ShapeDtypeStruct((N, D), F32),
                   jax.ShapeDtypeStruct((N, 1), F32),
                   jax.ShapeDtypeStruct((N, 1), F32),
                   jax.ShapeDtypeStruct((N, 1), F32),
                   jax.ShapeDtypeStruct((1, 1), F32)],
    )(hacc, sacc, gate_bias.reshape(1, D), x1, *gru0, gat1_W,
      att_src.reshape(D, 1), att_dst.reshape(D, 1))


def _t4a_body(hacc_ref, sacc_ref, eself_ref, hsrc_ref, gbias_ref, x2_ref,
              wih_ref, whh_ref, bih_ref, bhh_ref, x3_ref):
    s2 = jnp.sum(sacc_ref[...], axis=0)[:, None] + eself_ref[...]
    hun = hacc_ref[0] + hacc_ref[1] + eself_ref[...] * hsrc_ref[...]
    h = _elu(hun / (s2 + 1e-16) + gbias_ref[...])
    x3_ref[...] = jnp.maximum(_gru(h, x2_ref[...], wih_ref[...], whh_ref[...],
                                   bih_ref[...], bhh_ref[...]), 0.0)


def _t4a(hacc2, sacc2, eself, hsrc, gat1_bias, x2, gru1):
    return pl.pallas_call(
        _t4a_body,
        out_shape=jax.ShapeDtypeStruct((N, D), F32),
    )(hacc2, sacc2, eself, hsrc, gat1_bias.reshape(1, D), x2, *gru1)


def _t4b_body(x3_ref, batch_ref, wsrc_ref, wdst_ref, attsrc_ref, attdst_ref,
              mbias_ref, wih_ref, whh_ref, bih_ref, bhh_ref, out_ref):
    x3 = x3_ref[...]
    seg = lax.broadcasted_iota(jnp.int32, (G, N), 0)
    mask = (batch_ref[...] == seg).astype(F32)
    out = jnp.maximum(jnp.dot(mask, x3, preferred_element_type=F32), 0.0)
    for _ in range(L_MOL):
        hs = jnp.dot(x3, wsrc_ref[...].T, preferred_element_type=F32)
        hd = jnp.dot(out, wdst_ref[...].T, preferred_element_type=F32)
        asrc = jnp.dot(hs, attsrc_ref[...], preferred_element_type=F32)
        adst = jnp.dot(hd, attdst_ref[...], preferred_element_type=F32)
        adst_n = jnp.dot(adst.T, mask, preferred_element_type=F32).T
        logits = _leaky(asrc + adst_n, 0.2)
        lg = jnp.where(mask > 0, logits.T, -jnp.inf)
        m = jnp.max(lg, axis=1, keepdims=True)
        m = jnp.where(jnp.isfinite(m), m, 0.0)
        m_n = jnp.dot(m.T, mask, preferred_element_type=F32).T
        ee = jnp.exp(logits - m_n)
        sseg = jnp.dot(mask, ee, preferred_element_type=F32)
        s_n = jnp.dot(sseg.T, mask, preferred_element_type=F32).T
        alpha = ee / (s_n + 1e-16)
        hm = _elu(jnp.dot(mask, hs * alpha, preferred_element_type=F32)
                  + mbias_ref[...])
        out = jnp.maximum(_gru(hm, out, wih_ref[...], whh_ref[...],
                               bih_ref[...], bhh_ref[...]), 0.0)
    out_ref[...] = out


def _t4b(x3, batch, p):
    return pl.pallas_call(
        _t4b_body,
        out_shape=jax.ShapeDtypeStruct((G, D), F32),
    )(x3, batch.reshape(1, N), p['mol_W_src'], p['mol_W_dst'],
      p['mol_att_src'].reshape(D, 1), p['mol_att_dst'].reshape(D, 1),
      p['mol_bias'].reshape(1, D),
      p['mol_gru_Wih'], p['mol_gru_Whh'],
      p['mol_gru_bih'].reshape(1, 3 * H), p['mol_gru_bhh'].reshape(1, 3 * H))


def kernel(x, edge_index, edge_attr, batch, params):
    p = params
    src, dst = edge_index[0], edge_index[1]
    W1 = p['gate_lin1_W'][:, :H]
    W2 = p['gate_lin1_W'][:, H:]

    x1, xproj, ar, armax = _t1(x, p['lin1_W'], p['lin1_b'].reshape(1, D),
                               W1, p['gate_att_r'])

    gsrc = _sc_gather(xproj, src)

    mmsg, xjatt, xjattmax = _t2(edge_attr, gsrc, W2, p['gate_lin1_b'],
                                p['gate_lin2_W'], p['gate_att_l'])

    b1 = _leaky(xjattmax[0, 0] + armax[0, 0], 0.01)
    sacc, hacc = _sc_aggregate(
        0.01, False, mmsg, xjatt.reshape(E), dst,
        ar.reshape(N), ar.reshape(N), jnp.full((LN,), b1, F32))

    gru0 = (p['gru0_Wih'], p['gru0_Whh'],
            p['gru0_bih'].reshape(1, 3 * H), p['gru0_bhh'].reshape(1, 3 * H))
    x2, hsrc, as_, ad, eself, b2 = _t3(hacc, sacc, p['gate_bias'], x1, gru0,
                                       p['gat1_W'], p['gat1_att_src'],
                                       p['gat1_att_dst'])

    sacc2, hacc2 = _sc_aggregate(
        0.2, True, hsrc, src, dst,
        as_.reshape(N), ad.reshape(N),
        jnp.full((LN,), b2[0, 0], F32))

    gru1 = (p['gru1_Wih'], p['gru1_Whh'],
            p['gru1_bih'].reshape(1, 3 * H), p['gru1_bhh'].reshape(1, 3 * H))
    x3 = _t4a(hacc2, sacc2, eself, hsrc, p['gat1_bias'], x2, gru1)

    return _t4b(x3, batch, p)


# speedup vs baseline: 19.6776x; 1.0057x over previous
import functools

import jax
import jax.numpy as jnp
from jax import lax
from jax.experimental import pallas as pl
from jax.experimental.pallas import tpu as pltpu
from jax.experimental.pallas import tpu_sc as plsc

N = 10000
E = 320000
D = 128
H = 128
G = 64
L_MOL = 2

NC, NS, LN = 2, 16, 16
NW = NC * NS
EW = E // NW
CB = 80
NB = EW // CB
BE = 3200
NBE = E // BE

F32 = jnp.float32


def _leaky(x, s):
    return jnp.where(x >= 0, x, s * x)


def _sigmoid(x):
    return 1.0 / (1.0 + jnp.exp(-x))


def _elu(x):
    return jnp.where(x > 0, x, jnp.exp(jnp.minimum(x, 0.0)) - 1.0)


def _gru(inp, h, Wih, Whh, bih, bhh):
    gi = jnp.dot(inp, Wih.T, preferred_element_type=F32) + bih
    gh = jnp.dot(h, Whh.T, preferred_element_type=F32) + bhh
    r = _sigmoid(gi[:, :H] + gh[:, :H])
    z = _sigmoid(gi[:, H:2 * H] + gh[:, H:2 * H])
    n = jnp.tanh(gi[:, 2 * H:] + r * gh[:, 2 * H:])
    return (1.0 - z) * n + z * h


def _t1_body(x_ref, w_ref, b_ref, w1_ref, attr_ref,
             x1_ref, xproj_ref, ar_ref, armax_ref):
    x1 = _leaky(jnp.dot(x_ref[...], w_ref[...].T, preferred_element_type=F32)
                + b_ref[...], 0.01)
    x1_ref[...] = x1
    xproj_ref[...] = jnp.dot(x1, w1_ref[...].T, preferred_element_type=F32)
    ar = jnp.dot(x1, attr_ref[...], preferred_element_type=F32)
    ar_ref[...] = ar
    armax_ref[...] = jnp.max(ar).reshape(1, 1)


def _t1(x, lin1_W, lin1_b, W1, att_r):
    return pl.pallas_call(
        _t1_body,
        out_shape=[jax.ShapeDtypeStruct((N, D), F32),
                   jax.ShapeDtypeStruct((N, D), F32),
                   jax.ShapeDtypeStruct((N, 1), F32),
                   jax.ShapeDtypeStruct((1, 1), F32)],
    )(x, lin1_W, lin1_b, W1, att_r.reshape(D, 1))


def _t2_body(ea_ref, gsrc_ref, w2_ref, bg_ref, attl_ref,
             xj_ref, xjatt_ref, xjattmax_ref):
    i = pl.program_id(0)
    xj = _leaky(gsrc_ref[...]
                + jnp.dot(ea_ref[...], w2_ref[...].T, preferred_element_type=F32)
                + bg_ref[...], 0.01)
    xj_ref[...] = xj
    xa = jnp.dot(xj, attl_ref[...], preferred_element_type=F32)
    xjatt_ref[...] = xa

    @pl.when(i == 0)
    def _():
        xjattmax_ref[...] = jnp.full((1, 1), -jnp.inf, F32)
    xjattmax_ref[...] = jnp.maximum(xjattmax_ref[...], jnp.max(xa).reshape(1, 1))


def _t2(edge_attr, gsrc, W2, bg, att_l):
    return pl.pallas_call(
        _t2_body,
        grid=(NBE,),
        in_specs=[pl.BlockSpec((BE, D), lambda i: (i, 0)),
                  pl.BlockSpec((BE, D), lambda i: (i, 0)),
                  pl.BlockSpec((D, D), lambda i: (0, 0)),
                  pl.BlockSpec((1, D), lambda i: (0, 0)),
                  pl.BlockSpec((D, 1), lambda i: (0, 0))],
        out_specs=[pl.BlockSpec((BE, D), lambda i: (i, 0)),
                   pl.BlockSpec((BE, 1), lambda i: (i, 0)),
                   pl.BlockSpec((1, 1), lambda i: (0, 0))],
        out_shape=[jax.ShapeDtypeStruct((E, D), F32),
                   jax.ShapeDtypeStruct((E, 1), F32),
                   jax.ShapeDtypeStruct((1, 1), F32)],
    )(edge_attr, gsrc, W2, bg.reshape(1, D), att_l.reshape(D, 1))


CBG = 400
NBG = EW // CBG


def _scg_body(tab_hbm, idx_hbm, out_hbm,
              idx_v0, idx_v1, rows_v0, rows_v1,
              semi0, semi1, semg, semo0, semo1):
    wid = lax.axis_index("s") * NC + lax.axis_index("c")
    base0 = wid * EW
    idx_b = (idx_v0, idx_v1)
    rows_b = (rows_v0, rows_v1)
    semi = (semi0, semi1)
    semo = (semo0, semo1)

    for b in range(2):
        pltpu.async_copy(idx_hbm.at[pl.ds(base0 + b * CBG, CBG)],
                         idx_b[b], semi[b])

    def block(g, b, first, prefetch):
        if not first:
            pltpu.make_async_copy(
                rows_b[b], out_hbm.at[pl.ds(base0 + (g - 2) * CBG, CBG)],
                semo[b]).wait()
        pltpu.make_async_copy(idx_hbm.at[pl.ds(base0 + g * CBG, CBG)],
                              idx_b[b], semi[b]).wait()
        pltpu.async_copy(tab_hbm.at[idx_b[b]], rows_b[b], semg).wait()

        if prefetch == "dynamic":
            @pl.when(g + 2 <= NBG - 1)
            def _():
                pltpu.async_copy(idx_hbm.at[pl.ds(base0 + (g + 2) * CBG, CBG)],
                                 idx_b[b], semi[b])
        elif prefetch:
            pltpu.async_copy(idx_hbm.at[pl.ds(base0 + (g + 2) * CBG, CBG)],
                             idx_b[b], semi[b])
        pltpu.async_copy(rows_b[b], out_hbm.at[pl.ds(base0 + g * CBG, CBG)],
                         semo[b])

    block(0, 0, True, True)
    block(1, 1, True, True)

    def pair(i, _):
        block(2 * i, 0, False, "dynamic")
        block(2 * i + 1, 1, False, "dynamic")
        return 0
    lax.fori_loop(1, NBG // 2, pair, 0, unroll=False)
    block(NBG - 1, 0, False, False)

    pltpu.make_async_copy(
        rows_b[0], out_hbm.at[pl.ds(base0 + (NBG - 1) * CBG, CBG)],
        semo[0]).wait()
    pltpu.make_async_copy(
        rows_b[1], out_hbm.at[pl.ds(base0 + (NBG - 2) * CBG, CBG)],
        semo[1]).wait()


def _sc_gather(table, idx):
    mesh = plsc.VectorSubcoreMesh(core_axis_name="c", subcore_axis_name="s",
                                  num_cores=NC, num_subcores=NS)
    return pl.kernel(
        _scg_body,
        out_type=jax.ShapeDtypeStruct((E, D), F32),
        mesh=mesh,
        scratch_types=[pltpu.VMEM((CBG,), jnp.int32),
                       pltpu.VMEM((CBG,), jnp.int32),
                       pltpu.VMEM((CBG, D), F32),
                       pltpu.VMEM((CBG, D), F32),
                       pltpu.SemaphoreType.DMA,
                       pltpu.SemaphoreType.DMA,
                       pltpu.SemaphoreType.DMA,
                       pltpu.SemaphoreType.DMA,
                       pltpu.SemaphoreType.DMA],
    )(table, idx)


def _sc_agg_body(slope, gather_rows,
                 rowsrc_hbm, aux_hbm, dst_hbm, ars_hbm, ard_hbm,
                 b_hbm, zeros_hbm, zeros1_hbm,
                 sacc_hbm, hacc_hbm,
                 ars_v, ard_v, b_v,
                 dst_v0, dst_v1, aux_v0, aux_v1, e_v0, e_v1,
                 rows_v0, rows_v1,
                 shared, shared_s,
                 semi0, semi1, semg, sems0, sems1):
    c = lax.axis_index("c")
    s = lax.axis_index("s")
    wid = s * NC + c
    base0 = wid * EW
    dst_b = (dst_v0, dst_v1)
    aux_b = (aux_v0, aux_v1)
    e_b = (e_v0, e_v1)
    rows_b = (rows_v0, rows_v1)
    semi = (semi0, semi1)
    sems = (sems0, sems1)

    pltpu.sync_copy(ard_hbm, ard_v)
    if gather_rows:
        pltpu.sync_copy(ars_hbm, ars_v)
    pltpu.sync_copy(b_hbm, b_v)

    @pl.when(s == 0)
    def _():
        pltpu.sync_copy(zeros_hbm, shared)
        pltpu.sync_copy(zeros1_hbm, shared_s)
    plsc.subcore_barrier()

    bv = b_v[...]

    def block(g, b, first):
        if not first:
            pltpu.make_async_copy(rows_b[b], shared.at[dst_b[b]],
                                  sems[b]).wait()
            pltpu.make_async_copy(e_b[b], shared_s.at[dst_b[b]],
                                  sems[b]).wait()
        base = base0 + g * CB
        cps = [pltpu.async_copy(dst_hbm.at[pl.ds(base, CB)], dst_b[b],
                                semi[b]),
               pltpu.async_copy(aux_hbm.at[pl.ds(base, CB)], aux_b[b],
                                semi[b])]
        if not gather_rows:
            cps.append(pltpu.async_copy(rowsrc_hbm.at[pl.ds(base, CB)],
                                        rows_b[b], semi[b]))
        for cp in cps:
            cp.wait()
        if gather_rows:
            pltpu.async_copy(rowsrc_hbm.at[aux_b[b]], rows_b[b], semg).wait()

        def inner(j, _):
            dvec = dst_b[b][pl.ds(j * LN, LN)]
            adv = plsc.load_gather(ard_v, [dvec])
            av = aux_b[b][pl.ds(j * LN, LN)]
            if gather_rows:
                t = plsc.load_gather(ars_v, [av]) + adv
            else:
                t = av + adv
            logit = jnp.where(t >= 0, t, slope * t)
            e = jnp.exp(logit - bv)
            e_b[b][pl.ds(j * LN, LN)] = e
            for k in range(LN):
                wv = jnp.zeros((LN,), F32) + e[k]
                r = j * LN + k
                for q in range(D // LN):
                    rows_b[b][r, pl.ds(q * LN, LN)] = (
                        rows_b[b][r, pl.ds(q * LN, LN)] * wv)
            return 0
        lax.fori_loop(0, CB // LN, inner, 0, unroll=False)

        pltpu.async_copy(rows_b[b], shared.at[dst_b[b]], sems[b], add=True)
        pltpu.async_copy(e_b[b], shared_s.at[dst_b[b]], sems[b], add=True)

    block(0, 0, True)
    block(1, 1, True)

    def pairs(i, _):
        block(2 * i, 0, False)
        block(2 * i + 1, 1, False)
        return 0
    lax.fori_loop(1, NB // 2, pairs, 0, unroll=False)
    block(NB - 1, 0, False)

    for b in range(2):
        pltpu.make_async_copy(rows_b[b], shared.at[dst_b[b]],
                              sems[b]).wait()
        pltpu.make_async_copy(e_b[b], shared_s.at[dst_b[b]],
                              sems[b]).wait()

    plsc.subcore_barrier()

    @pl.when(s == 0)
    def _():
        pltpu.sync_copy(shared, hacc_hbm.at[c])
        pltpu.sync_copy(shared_s, sacc_hbm.at[c])


def _sc_aggregate(slope, gather_rows, rowsrc, aux, dst, ars, ard, bound):
    mesh = plsc.VectorSubcoreMesh(core_axis_name="c", subcore_axis_name="s",
                                  num_cores=NC, num_subcores=NS)
    body = functools.partial(_sc_agg_body, slope, gather_rows)
    aux_dt = jnp.int32 if gather_rows else F32
    return pl.kernel(
        body,
        out_type=[jax.ShapeDtypeStruct((NC, N), F32),
                  jax.ShapeDtypeStruct((NC, N, D), F32)],
        mesh=mesh,
        compiler_params=pltpu.CompilerParams(use_tc_tiling_on_sc=False,
                                             needs_layout_passes=False),
        scratch_types=[pltpu.VMEM((N if gather_rows else LN,), F32),
                       pltpu.VMEM((N,), F32),
                       pltpu.VMEM((LN,), F32),
                       pltpu.VMEM((CB,), jnp.int32),
                       pltpu.VMEM((CB,), jnp.int32),
                       pltpu.VMEM((CB,), aux_dt),
                       pltpu.VMEM((CB,), aux_dt),
                       pltpu.VMEM((CB,), F32),
                       pltpu.VMEM((CB,), F32),
                       pltpu.VMEM((CB, D), F32),
                       pltpu.VMEM((CB, D), F32),
                       pltpu.VMEM_SHARED((N, D), F32),
                       pltpu.VMEM_SHARED((N,), F32),
                       pltpu.SemaphoreType.DMA,
                       pltpu.SemaphoreType.DMA,
                       pltpu.SemaphoreType.DMA,
                       pltpu.SemaphoreType.DMA,
                       pltpu.SemaphoreType.DMA],
    )(rowsrc, aux, dst, ars, ard, bound,
      jnp.zeros((N, D), F32), jnp.zeros((N,), F32))


def _t3_body(hacc_ref, sacc_ref, wl2_ref, gbias_ref, x1_ref,
             wih_ref, whh_ref, bih_ref, bhh_ref,
             gat1w_ref, attsrc_ref, attdst_ref,
             x2_ref, as_ref, ad_ref, eself_ref, b2_ref):
    ssum = jnp.sum(sacc_ref[...], axis=0)[:, None]
    hx = (hacc_ref[0] + hacc_ref[1]) / (ssum + 1e-16)
    hsum = jnp.dot(hx, wl2_ref[...].T, preferred_element_type=F32)
    h = _elu(hsum + gbias_ref[...])
    x2 = jnp.maximum(_gru(h, x1_ref[...], wih_ref[...], whh_ref[...],
                          bih_ref[...], bhh_ref[...]), 0.0)
    x2_ref[...] = x2
    casrc = jnp.dot(gat1w_ref[...].T, attsrc_ref[...],
                    preferred_element_type=F32)
    cadst = jnp.dot(gat1w_ref[...].T, attdst_ref[...],
                    preferred_element_type=F32)
    as_ = jnp.dot(x2, casrc, preferred_element_type=F32)
    ad = jnp.dot(x2, cadst, preferred_element_type=F32)
    as_ref[...] = as_
    ad_ref[...] = ad
    b2 = _leaky(jnp.max(as_) + jnp.max(ad), 0.2)
    b2_ref[...] = b2.reshape(1, 1)
    eself_ref[...] = jnp.exp(_leaky(as_ + ad, 0.2) - b2)


def _t3(hacc, sacc, Wl2, gate_bias, x1, gru0, gat1_W, att_src, att_dst):
    return pl.pallas_call(
        _t3_body,
        out_shape=[jax.ShapeDtypeStruct((N, D), F32),
                   jax.ShapeDtypeStruct((N, 1), F32),
                   jax.ShapeDtypeStruct((N, 1), F32),
                   jax.ShapeDtypeStruct((N, 1), F32),
                   jax.ShapeDtypeStruct((1, 1), F32)],
    )(hacc, sacc, Wl2, gate_bias.reshape(1, D), x1, *gru0, gat1_W,
      att_src.reshape(D, 1), att_dst.reshape(D, 1))


def _t4_body(hacc_ref, sacc_ref, eself_ref, gat1w_ref, gbias_ref, x2_ref,
             gwih_ref, gwhh_ref, gbih_ref, gbhh_ref,
             batch_ref, wsrc_ref, wdst_ref, attsrc_ref, attdst_ref,
             mbias_ref, wih_ref, whh_ref, bih_ref, bhh_ref, out_ref):
    x2 = x2_ref[...]
    s2 = jnp.sum(sacc_ref[...], axis=0)[:, None] + eself_ref[...]
    hun = hacc_ref[0] + hacc_ref[1] + eself_ref[...] * x2
    h = jnp.dot(hun / (s2 + 1e-16), gat1w_ref[...].T,
                preferred_element_type=F32)
    h = _elu(h + gbias_ref[...])
    x3 = jnp.maximum(_gru(h, x2, gwih_ref[...], gwhh_ref[...],
                          gbih_ref[...], gbhh_ref[...]), 0.0)
    seg = lax.broadcasted_iota(jnp.int32, (G, N), 0)
    mask = (batch_ref[...] == seg).astype(F32)
    out = jnp.maximum(jnp.dot(mask, x3, preferred_element_type=F32), 0.0)
    for _ in range(L_MOL):
        hs = jnp.dot(x3, wsrc_ref[...].T, preferred_element_type=F32)
        hd = jnp.dot(out, wdst_ref[...].T, preferred_element_type=F32)
        asrc = jnp.dot(hs, attsrc_ref[...], preferred_element_type=F32)
        adst = jnp.dot(hd, attdst_ref[...], preferred_element_type=F32)
        adst_n = jnp.dot(adst.T, mask, preferred_element_type=F32).T
        logits = _leaky(asrc + adst_n, 0.2)
        lg = jnp.where(mask > 0, logits.T, -jnp.inf)
        m = jnp.max(lg, axis=1, keepdims=True)
        m = jnp.where(jnp.isfinite(m), m, 0.0)
        m_n = jnp.dot(m.T, mask, preferred_element_type=F32).T
        ee = jnp.exp(logits - m_n)
        sseg = jnp.dot(mask, ee, preferred_element_type=F32)
        s_n = jnp.dot(sseg.T, mask, preferred_element_type=F32).T
        alpha = ee / (s_n + 1e-16)
        hm = _elu(jnp.dot(mask, hs * alpha, preferred_element_type=F32)
                  + mbias_ref[...])
        out = jnp.maximum(_gru(hm, out, wih_ref[...], whh_ref[...],
                               bih_ref[...], bhh_ref[...]), 0.0)
    out_ref[...] = out


def _t4(hacc2, sacc2, eself, gat1_W, gat1_bias, x2, gru1, batch, p):
    return pl.pallas_call(
        _t4_body,
        out_shape=jax.ShapeDtypeStruct((G, D), F32),
    )(hacc2, sacc2, eself, gat1_W, gat1_bias.reshape(1, D), x2, *gru1,
      batch.reshape(1, N), p['mol_W_src'], p['mol_W_dst'],
      p['mol_att_src'].reshape(D, 1), p['mol_att_dst'].reshape(D, 1),
      p['mol_bias'].reshape(1, D),
      p['mol_gru_Wih'], p['mol_gru_Whh'],
      p['mol_gru_bih'].reshape(1, 3 * H), p['mol_gru_bhh'].reshape(1, 3 * H))


def kernel(x, edge_index, edge_attr, batch, params):
    p = params
    src, dst = edge_index[0], edge_index[1]
    W1 = p['gate_lin1_W'][:, :H]
    W2 = p['gate_lin1_W'][:, H:]

    x1, xproj, ar, armax = _t1(x, p['lin1_W'], p['lin1_b'].reshape(1, D),
                               W1, p['gate_att_r'])

    gsrc = _sc_gather(xproj, src)

    xj, xjatt, xjattmax = _t2(edge_attr, gsrc, W2, p['gate_lin1_b'],
                              p['gate_att_l'])

    b1 = _leaky(xjattmax[0, 0] + armax[0, 0], 0.01)
    sacc, hacc = _sc_aggregate(
        0.01, False, xj, xjatt.reshape(E), dst,
        ar.reshape(N), ar.reshape(N), jnp.full((LN,), b1, F32))

    gru0 = (p['gru0_Wih'], p['gru0_Whh'],
            p['gru0_bih'].reshape(1, 3 * H), p['gru0_bhh'].reshape(1, 3 * H))
    x2, as_, ad, eself, b2 = _t3(hacc, sacc, p['gate_lin2_W'],
                                 p['gate_bias'], x1, gru0,
                                 p['gat1_W'], p['gat1_att_src'],
                                 p['gat1_att_dst'])

    sacc2, hacc2 = _sc_aggregate(
        0.2, True, x2, src, dst,
        as_.reshape(N), ad.reshape(N),
        jnp.full((LN,), b2[0, 0], F32))

    gru1 = (p['gru1_Wih'], p['gru1_Whh'],
            p['gru1_bih'].reshape(1, 3 * H), p['gru1_bhh'].reshape(1, 3 * H))
    return _t4(hacc2, sacc2, eself, p['gat1_W'], p['gat1_bias'], x2, gru1,
               batch, p)
